# baseline (speedup 1.0000x reference)
import functools

import jax
import jax.numpy as jnp
from jax import lax
from jax.experimental import pallas as pl
from jax.experimental.pallas import tpu as pltpu
from jax.experimental.pallas import tpu_sc as plsc

_NUM_K = 64
_TOTAL = 2048 * 512
_NC, _NS = 2, 16
_NW = _NC * _NS
_PER_W = _TOTAL // _NW
_LANES = 16

_mesh = plsc.VectorSubcoreMesh(
    core_axis_name="c", subcore_axis_name="s", num_cores=_NC, num_subcores=_NS
)


@functools.partial(
    pl.kernel,
    mesh=_mesh,
    out_type=jax.ShapeDtypeStruct((_TOTAL,), jnp.float32),
    scratch_types=[
        pltpu.VMEM((_PER_W,), jnp.float32),
        pltpu.VMEM((_PER_W,), jnp.float32),
        pltpu.VMEM((_PER_W,), jnp.float32),
        pltpu.SemaphoreType.DMA,
        pltpu.SemaphoreType.DMA,
        pltpu.SemaphoreType.DMA,
    ],
)
def _gaines_or_sc(in_hbm, out_hbm, acc, buf0, buf1, sem_a, sem0, sem1):
    wid = lax.axis_index("s") * _NC + lax.axis_index("c")
    base = wid * _PER_W

    bufs = (buf0, buf1)
    sems = (sem0, sem1)

    cp_acc = pltpu.async_copy(in_hbm.at[pl.ds(base, _PER_W)], acc, sem_a)
    pending = {}
    for k in (1, 2):
        pending[k] = pltpu.async_copy(
            in_hbm.at[pl.ds(k * _TOTAL + base, _PER_W)], bufs[k % 2], sems[k % 2]
        )
    cp_acc.wait()

    for k in range(1, _NUM_K):
        b = bufs[k % 2]
        pending[k].wait()

        @plsc.parallel_loop(0, _PER_W, _LANES, unroll=8)
        def _accum(i):
            plsc.addupdate(acc.at[pl.ds(i, _LANES)], b[pl.ds(i, _LANES)])

        nxt = k + 2
        if nxt < _NUM_K:
            pending[nxt] = pltpu.async_copy(
                in_hbm.at[pl.ds(nxt * _TOTAL + base, _PER_W)], b, sems[k % 2]
            )

    @plsc.parallel_loop(0, _PER_W, _LANES, unroll=8)
    def _threshold(i):
        v = acc[pl.ds(i, _LANES)]
        acc[pl.ds(i, _LANES)] = jnp.where(v > 0.0, 1.0, 0.0).astype(jnp.float32)

    pltpu.sync_copy(acc, out_hbm.at[pl.ds(base, _PER_W)])


def kernel(input):
    flat = input.reshape(_NUM_K * _TOTAL)
    out = _gaines_or_sc(flat)
    return out.reshape(2048, 512)


# speedup vs baseline: 2.0773x; 2.0773x over previous
import functools

import jax
import jax.numpy as jnp
from jax import lax
from jax.experimental import pallas as pl
from jax.experimental.pallas import tpu as pltpu
from jax.experimental.pallas import tpu_sc as plsc

_NUM_K = 64
_ROWS, _COLS = 2048, 512
_NC, _NS = 2, 16
_NW = _NC * _NS
_ROWS_W = _ROWS // _NW
_PER_W = _ROWS_W * _COLS
_LANES = 16

_mesh = plsc.VectorSubcoreMesh(
    core_axis_name="c", subcore_axis_name="s", num_cores=_NC, num_subcores=_NS
)


@functools.partial(
    pl.kernel,
    mesh=_mesh,
    out_type=jax.ShapeDtypeStruct((_ROWS, _COLS), jnp.float32),
    scratch_types=[
        pltpu.VMEM((_ROWS_W, _COLS), jnp.float32),
        pltpu.VMEM((_ROWS_W, _COLS), jnp.float32),
        pltpu.VMEM((_ROWS_W, _COLS), jnp.float32),
        pltpu.SemaphoreType.DMA,
        pltpu.SemaphoreType.DMA,
        pltpu.SemaphoreType.DMA,
    ],
)
def _gaines_or_sc(in_hbm, out_hbm, acc, buf0, buf1, sem_a, sem0, sem1):
    wid = lax.axis_index("s") * _NC + lax.axis_index("c")
    row0 = wid * _ROWS_W

    bufs = (buf0, buf1)
    sems = (sem0, sem1)

    cp_acc = pltpu.async_copy(
        in_hbm.at[0, pl.ds(row0, _ROWS_W), :], acc, sem_a
    )
    pending = {}
    for k in (1, 2):
        pending[k] = pltpu.async_copy(
            in_hbm.at[k, pl.ds(row0, _ROWS_W), :], bufs[k % 2], sems[k % 2]
        )
    cp_acc.wait()

    for k in range(1, _NUM_K):
        b = bufs[k % 2]
        pending[k].wait()

        @plsc.parallel_loop(0, _PER_W, _LANES, unroll=8)
        def _accum(i):
            r = lax.shift_right_logical(i, 9)
            c = pl.multiple_of(lax.bitwise_and(i, _COLS - 1), _LANES)
            plsc.addupdate(acc.at[r, pl.ds(c, _LANES)], b[r, pl.ds(c, _LANES)])

        nxt = k + 2
        if nxt < _NUM_K:
            pending[nxt] = pltpu.async_copy(
                in_hbm.at[nxt, pl.ds(row0, _ROWS_W), :], b, sems[k % 2]
            )

    @plsc.parallel_loop(0, _PER_W, _LANES, unroll=8)
    def _threshold(i):
        r = lax.shift_right_logical(i, 9)
        c = pl.multiple_of(lax.bitwise_and(i, _COLS - 1), _LANES)
        v = acc[r, pl.ds(c, _LANES)]
        acc[r, pl.ds(c, _LANES)] = jnp.where(v > 0.0, 1.0, 0.0).astype(
            jnp.float32
        )

    pltpu.sync_copy(acc, out_hbm.at[pl.ds(row0, _ROWS_W), :])


def kernel(input):
    return _gaines_or_sc(input)


# speedup vs baseline: 3.5841x; 1.7253x over previous
import functools

import jax
import jax.numpy as jnp
from jax import lax
from jax.experimental import pallas as pl
from jax.experimental.pallas import tpu as pltpu
from jax.experimental.pallas import tpu_sc as plsc

_NUM_K = 64
_ROWS, _COLS = 2048, 512
_NC, _NS = 2, 16
_NW = _NC * _NS
_LANES = 16

_SC_ROWS = 512
_TC_ROWS = _ROWS - _SC_ROWS
_ROWS_W = _SC_ROWS // _NW
_PER_W = _ROWS_W * _COLS
_TC_BLOCK = 128

_mesh = plsc.VectorSubcoreMesh(
    core_axis_name="c", subcore_axis_name="s", num_cores=_NC, num_subcores=_NS
)


@functools.partial(
    pl.kernel,
    mesh=_mesh,
    out_type=jax.ShapeDtypeStruct((_SC_ROWS, _COLS), jnp.float32),
    scratch_types=[
        pltpu.VMEM((_ROWS_W, _COLS), jnp.float32),
        pltpu.VMEM((_ROWS_W, _COLS), jnp.float32),
        pltpu.VMEM((_ROWS_W, _COLS), jnp.float32),
        pltpu.SemaphoreType.DMA,
        pltpu.SemaphoreType.DMA,
        pltpu.SemaphoreType.DMA,
    ],
)
def _gaines_or_sc(in_hbm, out_hbm, acc, buf0, buf1, sem_a, sem0, sem1):
    wid = lax.axis_index("s") * _NC + lax.axis_index("c")
    row0 = _TC_ROWS + wid * _ROWS_W

    bufs = (buf0, buf1)
    sems = (sem0, sem1)

    cp_acc = pltpu.async_copy(
        in_hbm.at[0, pl.ds(row0, _ROWS_W), :], acc, sem_a
    )
    pending = {}
    for k in (1, 2):
        pending[k] = pltpu.async_copy(
            in_hbm.at[k, pl.ds(row0, _ROWS_W), :], bufs[k % 2], sems[k % 2]
        )
    cp_acc.wait()

    for k in range(1, _NUM_K):
        b = bufs[k % 2]
        pending[k].wait()

        @plsc.parallel_loop(0, _PER_W, _LANES, unroll=8)
        def _accum(i):
            r = lax.shift_right_logical(i, 9)
            c = pl.multiple_of(lax.bitwise_and(i, _COLS - 1), _LANES)
            plsc.addupdate(acc.at[r, pl.ds(c, _LANES)], b[r, pl.ds(c, _LANES)])

        nxt = k + 2
        if nxt < _NUM_K:
            pending[nxt] = pltpu.async_copy(
                in_hbm.at[nxt, pl.ds(row0, _ROWS_W), :], b, sems[k % 2]
            )

    @plsc.parallel_loop(0, _PER_W, _LANES, unroll=8)
    def _threshold(i):
        r = lax.shift_right_logical(i, 9)
        c = pl.multiple_of(lax.bitwise_and(i, _COLS - 1), _LANES)
        v = acc[r, pl.ds(c, _LANES)]
        acc[r, pl.ds(c, _LANES)] = jnp.where(v > 0.0, 1.0, 0.0).astype(
            jnp.float32
        )

    pltpu.sync_copy(acc, out_hbm.at[pl.ds(wid * _ROWS_W, _ROWS_W), :])


def _tc_body(x_ref, o_ref):
    o_ref[...] = jnp.max(x_ref[...], axis=0)


_gaines_or_tc = pl.pallas_call(
    _tc_body,
    grid=(_TC_ROWS // _TC_BLOCK,),
    in_specs=[
        pl.BlockSpec((_NUM_K, _TC_BLOCK, _COLS), lambda i: (0, i, 0)),
    ],
    out_specs=pl.BlockSpec((_TC_BLOCK, _COLS), lambda i: (i, 0)),
    out_shape=jax.ShapeDtypeStruct((_TC_ROWS, _COLS), jnp.float32),
)


def kernel(input):
    sc_out = _gaines_or_sc(input)
    tc_out = _gaines_or_tc(input)
    return jnp.concatenate([tc_out, sc_out], axis=0)
